# baseline (speedup 1.0000x reference)
import functools

import jax
import jax.numpy as jnp
from jax import lax
from jax.experimental import pallas as pl
from jax.experimental.pallas import tpu as pltpu
from jax.experimental.pallas import tpu_sc as plsc

N = 262144
C = 256
NC = 2
NS = 16
L = 16
NW = NC * NS
PER_W = N // NW
CH = 16
NCHK = PER_W // CH
NBUF = 16
KSUB = CH // L


def _build_ppd_kernel():
    mesh = plsc.VectorSubcoreMesh(core_axis_name="c", subcore_axis_name="s")

    @functools.partial(
        pl.kernel,
        out_type=[
            jax.ShapeDtypeStruct((NC, L), jnp.float32),
            jax.ShapeDtypeStruct((NC, L), jnp.float32),
        ],
        mesh=mesh,
        compiler_params=pltpu.CompilerParams(needs_layout_passes=False),
        scratch_types=[
            pltpu.VMEM((PER_W,), jnp.int32),
            pltpu.VMEM((NBUF, CH, C), jnp.float32),
            pltpu.VMEM((2, L), jnp.float32),
            pltpu.VMEM((NS, 2, L), jnp.float32),
            pltpu.VMEM((L,), jnp.float32),
            pltpu.VMEM((L,), jnp.float32),
            pltpu.VMEM_SHARED((NS, 2, L), jnp.float32),
            pltpu.SemaphoreType.DMA((NBUF,)),
        ],
    )
    def ppd_kernel(logits_hbm, tgt_hbm, out_sum_hbm, out_cnt_hbm,
                   tgt_v, ring_v, part_v, red_v,
                   row_v, row2_v, shared, sems):
        cid = lax.axis_index("c")
        sid = lax.axis_index("s")
        wid = sid * NC + cid
        base = wid * PER_W

        pltpu.sync_copy(tgt_hbm.at[pl.ds(base, PER_W)], tgt_v)

        lane_it = lax.iota(jnp.int32, L)

        def issue(c, b):
            pltpu.async_copy(logits_hbm.at[pl.ds(base + c * CH, CH)],
                             ring_v.at[b], sems.at[b])

        def wait(c, b):
            pltpu.make_async_copy(logits_hbm.at[pl.ds(base + c * CH, CH)],
                                  ring_v.at[b], sems.at[b]).wait()

        def extract(c, b, acc, cnt):
            for kk in range(KSUB):
                t = tgt_v[pl.ds(c * CH + kk * L, L)]
                rowk = lane_it + (kk * L)
                x = plsc.load_gather(ring_v.at[b], [rowk, t])
                valid = t != 0
                d_ = 1.0 - x
                acc = acc + jnp.where(valid, d_ * d_, 0.0)
                cnt = cnt + jnp.where(valid, 1.0, 0.0)
            return acc, cnt

        for b in range(NBUF):
            issue(b, b)

        def pipe_body(gg, carry):
            acc, cnt = carry
            for b in range(NBUF):
                c = gg * NBUF + b
                wait(c, b)
                acc, cnt = extract(c, b, acc, cnt)

                @pl.when(c + NBUF < NCHK)
                def _():
                    issue(c + NBUF, b)

            return acc, cnt

        zero = jnp.zeros((L,), jnp.float32)
        acc, cnt = lax.fori_loop(0, NCHK // NBUF, pipe_body, (zero, zero),
                                 unroll=False)

        part_v[0, :] = acc
        part_v[1, :] = cnt
        pltpu.sync_copy(part_v, shared.at[sid])
        plsc.subcore_barrier()

        @pl.when(sid == 0)
        def _():
            pltpu.sync_copy(shared, red_v)
            s = jnp.zeros((L,), jnp.float32)
            c_ = jnp.zeros((L,), jnp.float32)
            for w in range(NS):
                s = s + red_v[w, 0, :]
                c_ = c_ + red_v[w, 1, :]
            s_tot = s[0]
            c_tot = c_[0]
            for i in range(1, L):
                s_tot = s_tot + s[i]
                c_tot = c_tot + c_[i]
            row_v[...] = jnp.full((L,), s_tot, jnp.float32)
            pltpu.sync_copy(row_v, out_sum_hbm.at[cid])
            row2_v[...] = jnp.full((L,), c_tot, jnp.float32)
            pltpu.sync_copy(row2_v, out_cnt_hbm.at[cid])

    return ppd_kernel


_PPD = _build_ppd_kernel()


@jax.jit
def kernel(contrast_logits, contrast_target):
    tgt = contrast_target.astype(jnp.int32)
    sums, cnts = _PPD(contrast_logits, tgt)
    return (sums[0, 0] + sums[1, 0]) / (cnts[0, 0] + cnts[1, 0])


# speedup vs baseline: 1.0145x; 1.0145x over previous
import functools

import jax
import jax.numpy as jnp
from jax import lax
from jax.experimental import pallas as pl
from jax.experimental.pallas import tpu as pltpu
from jax.experimental.pallas import tpu_sc as plsc

N = 262144
C = 256
NC = 2
NS = 16
L = 16
NW = NC * NS
PER_W = N // NW
CH = 64
NCHK = PER_W // CH
NBUF = 4
KSUB = CH // L


def _build_ppd_kernel():
    mesh = plsc.VectorSubcoreMesh(core_axis_name="c", subcore_axis_name="s")

    @functools.partial(
        pl.kernel,
        out_type=[
            jax.ShapeDtypeStruct((NC, L), jnp.float32),
            jax.ShapeDtypeStruct((NC, L), jnp.float32),
        ],
        mesh=mesh,
        compiler_params=pltpu.CompilerParams(needs_layout_passes=False),
        scratch_types=[
            pltpu.VMEM((PER_W,), jnp.int32),
            pltpu.VMEM((NBUF, CH, C), jnp.float32),
            pltpu.VMEM((2, L), jnp.float32),
            pltpu.VMEM((NS, 2, L), jnp.float32),
            pltpu.VMEM((L,), jnp.float32),
            pltpu.VMEM((L,), jnp.float32),
            pltpu.VMEM_SHARED((NS, 2, L), jnp.float32),
            pltpu.SemaphoreType.DMA((NBUF,)),
        ],
    )
    def ppd_kernel(logits_hbm, tgt_hbm, out_sum_hbm, out_cnt_hbm,
                   tgt_v, ring_v, part_v, red_v,
                   row_v, row2_v, shared, sems):
        cid = lax.axis_index("c")
        sid = lax.axis_index("s")
        wid = sid * NC + cid
        base = wid * PER_W

        pltpu.sync_copy(tgt_hbm.at[pl.ds(base, PER_W)], tgt_v)

        lane_it = lax.iota(jnp.int32, L)

        def issue(c, b):
            pltpu.async_copy(logits_hbm.at[pl.ds(base + c * CH, CH)],
                             ring_v.at[b], sems.at[b])

        def wait(c, b):
            pltpu.make_async_copy(logits_hbm.at[pl.ds(base + c * CH, CH)],
                                  ring_v.at[b], sems.at[b]).wait()

        def extract(c, b, acc, cnt):
            for kk in range(KSUB):
                t = tgt_v[pl.ds(c * CH + kk * L, L)]
                rowk = lane_it + (kk * L)
                x = plsc.load_gather(ring_v.at[b], [rowk, t])
                valid = t != 0
                d_ = 1.0 - x
                acc = acc + jnp.where(valid, d_ * d_, 0.0)
                cnt = cnt + jnp.where(valid, 1.0, 0.0)
            return acc, cnt

        for b in range(NBUF):
            issue(b, b)

        def pipe_body(gg, carry):
            acc, cnt = carry
            for b in range(NBUF):
                c = gg * NBUF + b
                wait(c, b)
                acc, cnt = extract(c, b, acc, cnt)

                @pl.when(c + NBUF < NCHK)
                def _():
                    issue(c + NBUF, b)

            return acc, cnt

        zero = jnp.zeros((L,), jnp.float32)
        acc, cnt = lax.fori_loop(0, NCHK // NBUF, pipe_body, (zero, zero),
                                 unroll=False)

        part_v[0, :] = acc
        part_v[1, :] = cnt
        pltpu.sync_copy(part_v, shared.at[sid])
        plsc.subcore_barrier()

        @pl.when(sid == 0)
        def _():
            pltpu.sync_copy(shared, red_v)
            s = jnp.zeros((L,), jnp.float32)
            c_ = jnp.zeros((L,), jnp.float32)
            for w in range(NS):
                s = s + red_v[w, 0, :]
                c_ = c_ + red_v[w, 1, :]
            s_tot = s[0]
            c_tot = c_[0]
            for i in range(1, L):
                s_tot = s_tot + s[i]
                c_tot = c_tot + c_[i]
            row_v[...] = jnp.full((L,), s_tot, jnp.float32)
            pltpu.sync_copy(row_v, out_sum_hbm.at[cid])
            row2_v[...] = jnp.full((L,), c_tot, jnp.float32)
            pltpu.sync_copy(row2_v, out_cnt_hbm.at[cid])

    return ppd_kernel


_PPD = _build_ppd_kernel()


@jax.jit
def kernel(contrast_logits, contrast_target):
    tgt = contrast_target.astype(jnp.int32)
    sums, cnts = _PPD(contrast_logits, tgt)
    return (sums[0, 0] + sums[1, 0]) / (cnts[0, 0] + cnts[1, 0])


# speedup vs baseline: 1.0927x; 1.0771x over previous
import functools

import jax
import jax.numpy as jnp
from jax import lax
from jax.experimental import pallas as pl
from jax.experimental.pallas import tpu as pltpu
from jax.experimental.pallas import tpu_sc as plsc

N = 262144
C = 256
NC = 2
NS = 16
L = 16
NW = NC * NS
PER_W = N // NW
CH = 32
NCHK = PER_W // CH
NBUF = 8
KSUB = CH // L


def _build_ppd_kernel():
    mesh = plsc.VectorSubcoreMesh(core_axis_name="c", subcore_axis_name="s")

    @functools.partial(
        pl.kernel,
        out_type=[
            jax.ShapeDtypeStruct((NC, L), jnp.float32),
            jax.ShapeDtypeStruct((NC, L), jnp.float32),
        ],
        mesh=mesh,
        compiler_params=pltpu.CompilerParams(needs_layout_passes=False),
        scratch_types=[
            pltpu.VMEM((PER_W,), jnp.int32),
            pltpu.VMEM((NBUF, CH, C), jnp.float32),
            pltpu.VMEM((2, L), jnp.float32),
            pltpu.VMEM((NS, 2, L), jnp.float32),
            pltpu.VMEM((L,), jnp.float32),
            pltpu.VMEM((L,), jnp.float32),
            pltpu.VMEM_SHARED((NS, 2, L), jnp.float32),
            pltpu.SemaphoreType.DMA((NBUF,)),
        ],
    )
    def ppd_kernel(logits_hbm, tgt_hbm, out_sum_hbm, out_cnt_hbm,
                   tgt_v, ring_v, part_v, red_v,
                   row_v, row2_v, shared, sems):
        cid = lax.axis_index("c")
        sid = lax.axis_index("s")
        wid = sid * NC + cid
        base = wid * PER_W

        pltpu.sync_copy(tgt_hbm.at[pl.ds(base, PER_W)], tgt_v)

        lane_it = lax.iota(jnp.int32, L)

        def issue(c, b):
            pltpu.async_copy(logits_hbm.at[pl.ds(base + c * CH, CH)],
                             ring_v.at[b], sems.at[b])

        def wait(c, b):
            pltpu.make_async_copy(logits_hbm.at[pl.ds(base + c * CH, CH)],
                                  ring_v.at[b], sems.at[b]).wait()

        def extract(c, b, acc, cnt):
            for kk in range(KSUB):
                t = tgt_v[pl.ds(c * CH + kk * L, L)]
                rowk = lane_it + (kk * L)
                x = plsc.load_gather(ring_v.at[b], [rowk, t])
                valid = t != 0
                d_ = 1.0 - x
                acc = acc + jnp.where(valid, d_ * d_, 0.0)
                cnt = cnt + jnp.where(valid, 1.0, 0.0)
            return acc, cnt

        for b in range(NBUF):
            issue(b, b)

        def pipe_body(gg, carry):
            acc, cnt = carry
            for b in range(NBUF):
                c = gg * NBUF + b
                wait(c, b)
                acc, cnt = extract(c, b, acc, cnt)

                @pl.when(c + NBUF < NCHK)
                def _():
                    issue(c + NBUF, b)

            return acc, cnt

        zero = jnp.zeros((L,), jnp.float32)
        acc, cnt = lax.fori_loop(0, NCHK // NBUF, pipe_body, (zero, zero),
                                 unroll=False)

        part_v[0, :] = acc
        part_v[1, :] = cnt
        pltpu.sync_copy(part_v, shared.at[sid])
        plsc.subcore_barrier()

        @pl.when(sid == 0)
        def _():
            pltpu.sync_copy(shared, red_v)
            s = jnp.zeros((L,), jnp.float32)
            c_ = jnp.zeros((L,), jnp.float32)
            for w in range(NS):
                s = s + red_v[w, 0, :]
                c_ = c_ + red_v[w, 1, :]
            s_tot = s[0]
            c_tot = c_[0]
            for i in range(1, L):
                s_tot = s_tot + s[i]
                c_tot = c_tot + c_[i]
            row_v[...] = jnp.full((L,), s_tot, jnp.float32)
            pltpu.sync_copy(row_v, out_sum_hbm.at[cid])
            row2_v[...] = jnp.full((L,), c_tot, jnp.float32)
            pltpu.sync_copy(row2_v, out_cnt_hbm.at[cid])

    return ppd_kernel


_PPD = _build_ppd_kernel()


@jax.jit
def kernel(contrast_logits, contrast_target):
    tgt = contrast_target.astype(jnp.int32)
    sums, cnts = _PPD(contrast_logits, tgt)
    return (sums[0, 0] + sums[1, 0]) / (cnts[0, 0] + cnts[1, 0])
